# Initial kernel scaffold; baseline (speedup 1.0000x reference)
import jax
import jax.numpy as jnp
from jax.experimental import pallas as pl


def kernel(hD, hE, edge_e2d, edge_d2e, W_llr, W_m1, b_m1, W_m2, b_m2, W_sb):
    raise NotImplementedError("write your pallas kernel here")


# speedup vs baseline: 4.4925x; 4.4925x over previous
import functools

import jax
import jax.numpy as jnp
from jax import lax
from jax.experimental import pallas as pl
from jax.experimental.pallas import tpu as pltpu
from jax.experimental.pallas import tpu_sc as plsc

N = 10000
H = 128
E = 160000
ALPHA = 0.1

NC, NS = 2, 16
NW = NC * NS
EP = 163840
RPW = EP // NW // 128
NP = 10112
RPS = NP // NS
TRASH = N


def _sc_scatter_body(W, tab, srcg, dstg, zrows, out, acc, idxs, idxd, rows):
    c = lax.axis_index("c")
    s = lax.axis_index("s")
    wid = c * NS + s
    pltpu.sync_copy(zrows, acc.at[pl.ds(s * RPS, RPS)])
    plsc.subcore_barrier()
    base = wid * RPW

    @pl.loop(0, RPW // 8)
    def _(i):
        r0 = base + i * 8
        pltpu.sync_copy(srcg.at[pl.ds(r0, 8)], idxs)
        pltpu.sync_copy(dstg.at[pl.ds(r0, 8)], idxd)
        for j in range(8):
            pltpu.sync_copy(tab.at[idxs.at[j]], rows)
            pltpu.sync_copy(rows, acc.at[idxd.at[j]], add=True)

    plsc.subcore_barrier()
    pltpu.sync_copy(acc.at[pl.ds(s * RPS, RPS)], out.at[c, pl.ds(s * RPS, RPS)])


def _sc_scatter(table, srcg, dstg, W):
    mesh = plsc.VectorSubcoreMesh(core_axis_name="c", subcore_axis_name="s")
    zrows = jnp.zeros((RPS, W), jnp.float32)
    k = pl.kernel(
        functools.partial(_sc_scatter_body, W),
        out_type=jax.ShapeDtypeStruct((NC, NP, W), jnp.float32),
        mesh=mesh,
        scratch_types=[
            pltpu.VMEM_SHARED((NP, W), jnp.float32),
            pltpu.VMEM((8, 128), jnp.int32),
            pltpu.VMEM((8, 128), jnp.int32),
            pltpu.VMEM((128, W), jnp.float32),
        ],
        compiler_params=pltpu.CompilerParams(use_tc_tiling_on_sc=False),
    )
    return k(table, srcg, dstg, zrows)


def _tc1_body(he_ref, wllr_ref, out_ref):
    h = he_ref[...]
    llr = jnp.dot(h, wllr_ref[...].T, preferred_element_type=jnp.float32)
    pt = jnp.clip(-jnp.tanh(llr * 0.5), -0.999, 0.999)
    lg = jnp.log(jnp.maximum(jnp.abs(pt), 1e-8))
    neg = (pt < 0).astype(jnp.float32)
    pad = jnp.zeros((h.shape[0], 14), jnp.float32)
    out_ref[...] = jnp.concatenate([lg, neg, pad], axis=1)


def _tc1(hE2, W_llr):
    return pl.pallas_call(
        _tc1_body,
        out_shape=jax.ShapeDtypeStruct((N, 16), jnp.float32),
    )(hE2, W_llr)


def _tc2_body(hd_ref, parts_ref, w1a_ref, w1b_ref, b1_ref, w2_ref, b2_ref,
              hdn_ref, tab_ref):
    la = parts_ref[0, :, 0:1] + parts_ref[1, :, 0:1]
    ncnt = parts_ref[0, :, 1:2] + parts_ref[1, :, 1:2]
    sign = 1.0 - 2.0 * (ncnt % 2.0)
    par = sign * jnp.exp(jnp.minimum(la, 20.0))
    par = jnp.where(jnp.isfinite(par), par, 0.0)
    h = hd_ref[...]
    pre = (jnp.dot(h, w1a_ref[...].T, preferred_element_type=jnp.float32)
           + par * w1b_ref[...] + b1_ref[...])
    mixed = (jnp.dot(jnp.maximum(pre, 0.0), w2_ref[...].T,
                     preferred_element_type=jnp.float32) + b2_ref[...])
    hdn = h + ALPHA * mixed
    hdn_ref[...] = hdn
    r = h.shape[0]
    tab_ref[...] = jnp.concatenate(
        [hdn, jnp.ones((r, 1), jnp.float32), jnp.zeros((r, 15), jnp.float32)],
        axis=1)


def _tc2(hD2, parts1, W1a, w1b, b1, W2, b2):
    R = 1000
    grid = (N // R,)
    return pl.pallas_call(
        _tc2_body,
        grid=grid,
        in_specs=[
            pl.BlockSpec((R, H), lambda i: (i, 0)),
            pl.BlockSpec((NC, R, 16), lambda i: (0, i, 0)),
            pl.BlockSpec((H, H), lambda i: (0, 0)),
            pl.BlockSpec((1, H), lambda i: (0, 0)),
            pl.BlockSpec((1, H), lambda i: (0, 0)),
            pl.BlockSpec((H, H), lambda i: (0, 0)),
            pl.BlockSpec((1, H), lambda i: (0, 0)),
        ],
        out_specs=[
            pl.BlockSpec((R, H), lambda i: (i, 0)),
            pl.BlockSpec((R, 144), lambda i: (i, 0)),
        ],
        out_shape=[
            jax.ShapeDtypeStruct((N, H), jnp.float32),
            jax.ShapeDtypeStruct((N, 144), jnp.float32),
        ],
    )(hD2, parts1, W1a, w1b, b1, W2, b2)


def _tc3_body(he_ref, parts_ref, wsb_ref, out_ref):
    s = parts_ref[0] + parts_ref[1]
    cnt = jnp.maximum(s[:, 128:129], 1.0)
    agg = jnp.dot(s[:, :128], wsb_ref[...].T,
                  preferred_element_type=jnp.float32) / cnt
    out_ref[...] = he_ref[...] + ALPHA * agg


def _tc3(hE2, parts3, W_sb):
    R = 1000
    return pl.pallas_call(
        _tc3_body,
        grid=(N // R,),
        in_specs=[
            pl.BlockSpec((R, H), lambda i: (i, 0)),
            pl.BlockSpec((NC, R, 144), lambda i: (0, i, 0)),
            pl.BlockSpec((H, H), lambda i: (0, 0)),
        ],
        out_specs=pl.BlockSpec((R, H), lambda i: (i, 0)),
        out_shape=jax.ShapeDtypeStruct((N, H), jnp.float32),
    )(hE2, parts3, W_sb)


def _pad_edges(src, dst):
    pad = EP - E
    srcg = jnp.concatenate([src, jnp.zeros((pad,), jnp.int32)]).reshape(EP // 128, 128)
    dstg = jnp.concatenate([dst, jnp.full((pad,), TRASH, jnp.int32)]).reshape(EP // 128, 128)
    return srcg, dstg


def kernel(hD, hE, edge_e2d, edge_d2e, W_llr, W_m1, b_m1, W_m2, b_m2, W_sb):
    hD2, hE2 = hD[0], hE[0]
    srcg1, dstg1 = _pad_edges(edge_e2d[0].astype(jnp.int32),
                              edge_e2d[1].astype(jnp.int32))
    srcg2, dstg2 = _pad_edges(edge_d2e[0].astype(jnp.int32),
                              edge_d2e[1].astype(jnp.int32))

    table1 = _tc1(hE2, W_llr)
    parts1 = _sc_scatter(table1, srcg1, dstg1, 16)
    hD_new2, table3 = _tc2(hD2, parts1[:, :N, :], W_m1[:, :H],
                           W_m1[:, H].reshape(1, H), b_m1.reshape(1, H),
                           W_m2, b_m2.reshape(1, H))
    parts3 = _sc_scatter(table3, srcg2, dstg2, 144)
    hE_new2 = _tc3(hE2, parts3[:, :N, :], W_sb)
    return hD_new2[None], hE_new2[None]
